# baseline (speedup 1.0000x reference)
import functools

import jax
import jax.numpy as jnp
from jax import lax
from jax.experimental import pallas as pl
from jax.experimental.pallas import tpu as pltpu
from jax.experimental.pallas import tpu_sc as plsc

B = 8
N = 4096
L = 16
NC = 2
NS = 16
NW = NC * NS
CHUNKS = NW // B
CHUNK = N // CHUNKS
SBLK = 8
NT = N // L
NSV = CHUNK // L

_mesh = plsc.VectorSubcoreMesh(
    core_axis_name="c", subcore_axis_name="s", num_cores=NC, num_subcores=NS
)

_GATHER_DNUMS = lax.GatherDimensionNumbers(
    offset_dims=(), collapsed_slice_dims=(0,), start_index_map=(0,)
)


def _lane_bcast(v, k):
    idx = jnp.full((L, 1), k, jnp.int32)
    return lax.gather(
        v, idx, _GATHER_DNUMS, (1,),
        mode=lax.GatherScatterMode.PROMISE_IN_BOUNDS,
    )


def _bf16_rne(x):
    u = plsc.bitcast(x, jnp.uint32)
    r = u + jnp.uint32(0x7FFF) + ((u >> jnp.uint32(16)) & jnp.uint32(1))
    return plsc.bitcast(r & jnp.uint32(0xFFFF0000), jnp.float32)


@functools.partial(
    pl.kernel,
    mesh=_mesh,
    compiler_params=pltpu.CompilerParams(needs_layout_passes=False),
    out_type=jax.ShapeDtypeStruct((NW, 2, L), jnp.float32),
    scratch_types=[
        pltpu.VMEM((3, CHUNK), jnp.float32),
        pltpu.VMEM((3, N), jnp.float32),
        pltpu.VMEM((4, N), jnp.float32),
        pltpu.VMEM((2, L), jnp.float32),
    ],
)
def _sc_knn_loss(src_hbm, tgt_hbm, out_hbm, src_v, tgt_v, rw_v, out_v):
    wid = lax.axis_index("s") * NC + lax.axis_index("c")
    b = wid // CHUNKS

    pltpu.sync_copy(src_hbm.at[wid], src_v)
    pltpu.sync_copy(tgt_hbm.at[b], tgt_v)

    inf_v = jnp.full((L,), jnp.inf, jnp.float32)
    zero_v = jnp.zeros((L,), jnp.float32)
    one_v = jnp.full((L,), 1.0, jnp.float32)
    iota_v = lax.iota(jnp.int32, L)
    big_i = jnp.full((L,), N, jnp.int32)
    zero_i = jnp.zeros((L,), jnp.int32)
    one_i = jnp.full((L,), 1, jnp.int32)
    two_i = jnp.full((L,), 2, jnp.int32)
    step_i = jnp.full((L,), L, jnp.int32)

    def prep_body(j, _):
        o = j * L
        tx = tgt_v[0, pl.ds(o, L)]
        ty = tgt_v[1, pl.ds(o, L)]
        tz = tgt_v[2, pl.ds(o, L)]
        rw_v[0, pl.ds(o, L)] = 2.0 * _bf16_rne(tx)
        rw_v[1, pl.ds(o, L)] = 2.0 * _bf16_rne(ty)
        rw_v[2, pl.ds(o, L)] = 2.0 * _bf16_rne(tz)
        w = (tx * tx + ty * ty) + tz * tz
        valid = (tx != 0.0) | (ty != 0.0) | (tz != 0.0)
        rw_v[3, pl.ds(o, L)] = jnp.where(valid, w, inf_v)
        return 0

    lax.fori_loop(0, NT, prep_body, 0)

    def cnt_body(q, cta):
        o = q * L
        sx = src_v[0, pl.ds(o, L)]
        sy = src_v[1, pl.ds(o, L)]
        sz = src_v[2, pl.ds(o, L)]
        valid = (sx != 0.0) | (sy != 0.0) | (sz != 0.0)
        return cta + jnp.where(valid, one_v, zero_v)

    cta = lax.fori_loop(0, NSV, cnt_body, zero_v)

    def grp_body(q, dacc):
        o = q * L
        sxv = src_v[0, pl.ds(o, L)]
        syv = src_v[1, pl.ds(o, L)]
        szv = src_v[2, pl.ds(o, L)]
        rsx = _bf16_rne(sxv)
        rsy = _bf16_rne(syv)
        rsz = _bf16_rne(szv)
        for half in range(L // SBLK):
            hb = half * SBLK
            bx = [_lane_bcast(rsx, hb + k) for k in range(SBLK)]
            by = [_lane_bcast(rsy, hb + k) for k in range(SBLK)]
            bz = [_lane_bcast(rsz, hb + k) for k in range(SBLK)]

            def tgt_body(j, carry):
                jvec = carry[0]
                mv = carry[1:1 + SBLK]
                mi = carry[1 + SBLK:]
                to = j * L
                rtx = rw_v[0, pl.ds(to, L)]
                rty = rw_v[1, pl.ds(to, L)]
                rtz = rw_v[2, pl.ds(to, L)]
                w = rw_v[3, pl.ds(to, L)]
                nmv = []
                nmi = []
                for k in range(SBLK):
                    v = w - ((bx[k] * rtx + by[k] * rty) + bz[k] * rtz)
                    cmp = v < mv[k]
                    nmv.append(jnp.where(cmp, v, mv[k]))
                    nmi.append(jnp.where(cmp, jvec, mi[k]))
                return (jvec + step_i, *nmv, *nmi)

            init = (iota_v,) + (inf_v,) * SBLK + (zero_i,) * SBLK
            res = lax.fori_loop(0, NT, tgt_body, init)
            mv = res[1:1 + SBLK]
            mi = res[1 + SBLK:]

            osx = src_v[0, pl.ds(o, L)]
            osy = src_v[1, pl.ds(o, L)]
            osz = src_v[2, pl.ds(o, L)]
            for k in range(SBLK):
                gm = -_lane_bcast(plsc.cummax(-mv[k]), L - 1)
                cand = jnp.where(mv[k] == gm, mi[k], big_i)
                idx = -_lane_bcast(plsc.cummax(-cand), L - 1)
                gx = plsc.load_gather(tgt_v, [zero_i, idx])
                gy = plsc.load_gather(tgt_v, [one_i, idx])
                gz = plsc.load_gather(tgt_v, [two_i, idx])
                ox = _lane_bcast(osx, hb + k)
                oy = _lane_bcast(osy, hb + k)
                oz = _lane_bcast(osz, hb + k)
                dx = ox - gx
                dy = oy - gy
                dz = oz - gz
                dd = (dx * dx + dy * dy) + dz * dz
                svalid = (ox != 0.0) | (oy != 0.0) | (oz != 0.0)
                dacc = dacc + jnp.where(svalid, dd, zero_v)
        return dacc

    dacc = lax.fori_loop(0, NSV, grp_body, zero_v)

    out_v[0, :] = dacc
    out_v[1, :] = cta
    pltpu.sync_copy(out_v, out_hbm.at[wid])


def kernel(source_point_cloud, target_point_cloud):
    srcT = source_point_cloud.astype(jnp.float32).transpose(0, 2, 1)
    tgtT = target_point_cloud.astype(jnp.float32).transpose(0, 2, 1)
    src_chunks = (
        srcT.reshape(B, 3, CHUNKS, CHUNK).transpose(0, 2, 1, 3).reshape(NW, 3, CHUNK)
    )
    part = _sc_knn_loss(src_chunks, tgtT)

    d2sum = part[:, 0, 0]
    cnt = part[:, 1, :].sum(-1)
    loss_b = d2sum.reshape(B, CHUNKS).sum(1) / (3.0 * cnt.reshape(B, CHUNKS).sum(1))
    return jnp.mean(loss_b)
